# Initial kernel scaffold; baseline (speedup 1.0000x reference)
import jax
import jax.numpy as jnp
from jax.experimental import pallas as pl


def kernel(voxel_maps, grid_positions, features):
    raise NotImplementedError("write your pallas kernel here")


# speedup vs baseline: 1.0164x; 1.0164x over previous
import functools

import jax
import jax.numpy as jnp
from jax import lax
from jax.experimental import pallas as pl
from jax.experimental.pallas import tpu as pltpu
from jax.experimental.pallas import tpu_sc as plsc

_NC = 2
_NS = 16
_NW = _NC * _NS
_IDX_W = 128


def _addr_gpf_body(gp_ref, addr_ref, gpf_ref, *, Z, Y, X, G, O):
    gp = gp_ref[...]
    b = gp[:, 0:1]
    zg = gp[:, 1:2]
    yg = gp[:, 2:3]
    xg = gp[:, 3:4]
    o = lax.broadcasted_iota(jnp.int32, (G, O), 1)
    zo = (o & 3) - 2
    yo = ((o >> 2) & 3) - 2
    xo = (o >> 4) - 2
    z = jnp.clip(zg + zo, 0, Z - 1)
    y = jnp.clip(yg + yo, 0, Y - 1)
    x = jnp.clip(xg + xo, 0, X - 1)
    addr_ref[...] = ((b * Z + z) * Y + y) * X + x

    oo = lax.broadcasted_iota(jnp.int32, (4, G, O), 2)
    dd = lax.broadcasted_iota(jnp.int32, (4, G, O), 0)
    zo3 = (oo & 3) - 2
    yo3 = ((oo >> 2) & 3) - 2
    xo3 = (oo >> 4) - 2
    gpf_ref[...] = jnp.where(
        dd == 1, zo3, jnp.where(dd == 2, yo3, jnp.where(dd == 3, xo3, 0))
    )


def _gather_body(voxel_hbm, feat_hbm, addr_hbm, inter_hbm, sidx_hbm,
                 addr_v, idx_v, rows_v, sem1, sem2, *, n_chunks):
    wid = lax.axis_index("s") * _NC + lax.axis_index("c")
    pltpu.sync_copy(addr_hbm.at[wid], addr_v)

    def j_body(j, carry):
        pltpu.async_copy(voxel_hbm.at[addr_v.at[j]], idx_v.at[j], sem1).wait()
        pltpu.async_copy(feat_hbm.at[idx_v.at[j]], rows_v, sem2).wait()
        pltpu.sync_copy(rows_v, inter_hbm.at[wid, j])
        return carry

    lax.fori_loop(0, n_chunks, j_body, 0)
    pltpu.sync_copy(idx_v, sidx_hbm.at[wid])


def _transpose_body(in_ref, out_ref):
    out_ref[...] = in_ref[...].T


def _mask_body(sidx_ref, mask_ref):
    s = jnp.sum(sidx_ref[...] + 1, axis=1, keepdims=True)
    mask_ref[...] = (s == 0).astype(jnp.int32)


def kernel(voxel_maps, grid_positions, features):
    N, Z, Y, X = voxel_maps.shape
    G = grid_positions.shape[0]
    O = 64
    F, C = features.shape
    B = G * O
    per_w = B // _NW
    n_chunks = per_w // _IDX_W

    addr, gpf = pl.pallas_call(
        functools.partial(_addr_gpf_body, Z=Z, Y=Y, X=X, G=G, O=O),
        out_shape=(
            jax.ShapeDtypeStruct((G, O), jnp.int32),
            jax.ShapeDtypeStruct((4, G, O), jnp.int32),
        ),
    )(grid_positions)
    addr3 = addr.reshape(_NW, n_chunks, _IDX_W)

    voxel_flat = voxel_maps.reshape(N * Z * Y * X)
    mesh = plsc.VectorSubcoreMesh(core_axis_name="c", subcore_axis_name="s")
    inter, sidx = pl.kernel(
        functools.partial(_gather_body, n_chunks=n_chunks),
        out_type=(
            jax.ShapeDtypeStruct((_NW, n_chunks, _IDX_W, C), jnp.float32),
            jax.ShapeDtypeStruct((_NW, n_chunks, _IDX_W), jnp.int32),
        ),
        mesh=mesh,
        compiler_params=pltpu.CompilerParams(use_tc_tiling_on_sc=False),
        scratch_types=[
            pltpu.VMEM((n_chunks, _IDX_W), jnp.int32),
            pltpu.VMEM((n_chunks, _IDX_W), jnp.int32),
            pltpu.VMEM((_IDX_W, C), jnp.float32),
            pltpu.SemaphoreType.DMA,
            pltpu.SemaphoreType.DMA,
        ],
    )(voxel_flat, features, addr3)

    inter2 = inter.reshape(B, C)
    blk = 2048
    out_t = pl.pallas_call(
        _transpose_body,
        out_shape=jax.ShapeDtypeStruct((C, B), jnp.float32),
        grid=(B // blk,),
        in_specs=[pl.BlockSpec((blk, C), lambda k: (k, 0))],
        out_specs=pl.BlockSpec((C, blk), lambda k: (0, k)),
    )(inter2)
    sampled_features = out_t.reshape(1, C, G, O)

    sidx2 = sidx.reshape(G, O)
    mask_i32 = pl.pallas_call(
        _mask_body,
        out_shape=jax.ShapeDtypeStruct((G, 1), jnp.int32),
    )(sidx2)
    empty_mask = mask_i32.reshape(G).astype(jnp.bool_)

    return (sampled_features, gpf.reshape(1, 4, G, O), empty_mask)
